# baseline (speedup 1.0000x reference)
import functools

import jax
import jax.numpy as jnp
from jax.experimental import pallas as pl
from jax.experimental.pallas import tpu as pltpu

_BK = 8000


def _nn_kernel(p_ref, c_ref, val_ref, idx_ref, *, bk, num_rows):
    i = pl.program_id(0)

    @pl.when(i == 0)
    def _init():
        val_ref[...] = jnp.full_like(val_ref, jnp.inf)
        idx_ref[...] = jnp.zeros_like(idx_ref)

    p = p_ref[...]
    c = c_ref[...]
    np_ = p.shape[0]
    dots = jax.lax.dot_general(
        c, -2.0 * p, (((1,), (1,)), ((), ())),
        preferred_element_type=jnp.float32,
        precision=jax.lax.Precision.HIGHEST,
    )
    c2b = jax.lax.dot_general(
        c * c, jnp.ones((p.shape[1], np_), jnp.float32),
        (((1,), (0,)), ((), ())),
        preferred_element_type=jnp.float32,
        precision=jax.lax.Precision.HIGHEST,
    )
    scores = dots + c2b

    local_min = jnp.min(scores, axis=0, keepdims=True)
    row_ids = jax.lax.broadcasted_iota(jnp.int32, scores.shape, 0)
    masked = jnp.where(scores == local_min, row_ids, num_rows)
    local_arg = jnp.min(masked, axis=0, keepdims=True)

    prev_v = val_ref[...]
    prev_i = idx_ref[...]
    better = local_min < prev_v
    val_ref[...] = jnp.where(better, local_min, prev_v)
    idx_ref[...] = jnp.where(better, i * bk + local_arg, prev_i)


def kernel(prompt_embs, clip_embs):
    num_rows, d = clip_embs.shape
    p = prompt_embs.shape[0]
    bk = _BK
    grid = num_rows // bk

    val, idx = pl.pallas_call(
        functools.partial(_nn_kernel, bk=bk, num_rows=num_rows),
        grid=(grid,),
        in_specs=[
            pl.BlockSpec((p, d), lambda i: (0, 0)),
            pl.BlockSpec((bk, d), lambda i: (i, 0)),
        ],
        out_specs=[
            pl.BlockSpec((1, p), lambda i: (0, 0)),
            pl.BlockSpec((1, p), lambda i: (0, 0)),
        ],
        out_shape=[
            jax.ShapeDtypeStruct((1, p), jnp.float32),
            jax.ShapeDtypeStruct((1, p), jnp.int32),
        ],
    )(prompt_embs, clip_embs)

    ids = idx[0, :]
    return (prompt_embs, prompt_embs, ids)


# speedup vs baseline: 2.2839x; 2.2839x over previous
import functools

import jax
import jax.numpy as jnp
from jax.experimental import pallas as pl
from jax.experimental.pallas import tpu as pltpu

_BK = 8000


def _nn_kernel(p_ref, c_ref, val_ref, idx_ref, *, bk, num_rows):
    i = pl.program_id(0)

    @pl.when(i == 0)
    def _init():
        val_ref[...] = jnp.full_like(val_ref, jnp.inf)
        idx_ref[...] = jnp.zeros_like(idx_ref)

    p = p_ref[...]
    c = c_ref[...]
    np_ = p.shape[0]
    dots = jax.lax.dot_general(
        c, -2.0 * p, (((1,), (1,)), ((), ())),
        preferred_element_type=jnp.float32,
        precision=jax.lax.Precision.DEFAULT,
    )
    c2b = jax.lax.dot_general(
        c * c, jnp.ones((p.shape[1], np_), jnp.float32),
        (((1,), (0,)), ((), ())),
        preferred_element_type=jnp.float32,
        precision=jax.lax.Precision.DEFAULT,
    )
    scores = dots + c2b

    local_min = jnp.min(scores, axis=0, keepdims=True)
    row_ids = jax.lax.broadcasted_iota(jnp.int32, scores.shape, 0)
    masked = jnp.where(scores == local_min, row_ids, num_rows)
    local_arg = jnp.min(masked, axis=0, keepdims=True)

    prev_v = val_ref[...]
    prev_i = idx_ref[...]
    better = local_min < prev_v
    val_ref[...] = jnp.where(better, local_min, prev_v)
    idx_ref[...] = jnp.where(better, i * bk + local_arg, prev_i)


def kernel(prompt_embs, clip_embs):
    num_rows, d = clip_embs.shape
    p = prompt_embs.shape[0]
    bk = _BK
    grid = num_rows // bk

    val, idx = pl.pallas_call(
        functools.partial(_nn_kernel, bk=bk, num_rows=num_rows),
        grid=(grid,),
        in_specs=[
            pl.BlockSpec((p, d), lambda i: (0, 0)),
            pl.BlockSpec((bk, d), lambda i: (i, 0)),
        ],
        out_specs=[
            pl.BlockSpec((1, p), lambda i: (0, 0)),
            pl.BlockSpec((1, p), lambda i: (0, 0)),
        ],
        out_shape=[
            jax.ShapeDtypeStruct((1, p), jnp.float32),
            jax.ShapeDtypeStruct((1, p), jnp.int32),
        ],
    )(prompt_embs, clip_embs)

    ids = idx[0, :]
    return (prompt_embs, prompt_embs, ids)


# speedup vs baseline: 8.5333x; 3.7363x over previous
import functools

import jax
import jax.numpy as jnp
from jax.experimental import pallas as pl
from jax.experimental.pallas import tpu as pltpu

_BKL = 7808
_NBUF = 6


def _scores(w1, w2, ones, ct):
    ct_hi = ct.astype(jnp.bfloat16).astype(jnp.float32)
    ct_lo = ct - ct_hi
    sq = ct * ct
    sq_hi = sq.astype(jnp.bfloat16).astype(jnp.float32)
    sq_lo = sq - sq_hi

    def mm(a, b):
        return jax.lax.dot_general(
            a, b, (((1,), (0,)), ((), ())),
            preferred_element_type=jnp.float32)

    return (mm(w1, ct_hi) + mm(w1, ct_lo)) + (
        mm(w2, ct_hi) + (mm(ones, sq_hi) + mm(ones, sq_lo)))


def _argmin_lanes(scores, base, num_rows):
    local_min = jnp.min(scores, axis=1, keepdims=True)
    lane_ids = jax.lax.broadcasted_iota(jnp.int32, scores.shape, 1)
    masked = jnp.where(scores == local_min, lane_ids, num_rows)
    local_arg = jnp.min(masked, axis=1, keepdims=True)
    return local_min, base + local_arg


def _nn_kernel(w1_ref, w2_ref, tail_ref, ct_hbm, idx_ref, bufs, sems,
               val_s, idx_s, *, bkl, nbuf, nstep, num_rows, tail):
    i = pl.program_id(0)
    w1 = w1_ref[...]
    w2 = w2_ref[...]
    ones = jnp.ones_like(w1)

    @pl.when(i == 0)
    def _prologue():
        for b in range(min(nbuf, nstep)):
            pltpu.make_async_copy(
                ct_hbm.at[:, pl.ds(b * bkl, bkl)], bufs.at[b], sems.at[b]
            ).start()
        if tail:
            tv, ti = _argmin_lanes(
                _scores(w1, w2, ones, tail_ref[...]),
                nstep * bkl, num_rows)
            val_s[...] = tv
            idx_s[...] = ti
        else:
            val_s[...] = jnp.full_like(val_s, jnp.inf)
            idx_s[...] = jnp.zeros_like(idx_s)

    slot = jax.lax.rem(i, nbuf)
    pltpu.make_async_copy(
        ct_hbm.at[:, pl.ds(i * bkl, bkl)], bufs.at[slot], sems.at[slot]
    ).wait()

    local_min, local_idx = _argmin_lanes(
        _scores(w1, w2, ones, bufs[slot]), i * bkl, num_rows)

    prev_v = val_s[...]
    prev_i = idx_s[...]
    better = (local_min < prev_v) | (
        (local_min == prev_v) & (local_idx < prev_i))
    val_s[...] = jnp.where(better, local_min, prev_v)
    idx_s[...] = jnp.where(better, local_idx, prev_i)

    @pl.when(i + nbuf < nstep)
    def _refill():
        pltpu.make_async_copy(
            ct_hbm.at[:, pl.ds((i + nbuf) * bkl, bkl)], bufs.at[slot],
            sems.at[slot]
        ).start()

    @pl.when(i == nstep - 1)
    def _finish():
        idx_ref[...] = idx_s[...]


def kernel(prompt_embs, clip_embs):
    num_rows, d = clip_embs.shape
    p = prompt_embs.shape[0]
    bkl = _BKL
    nbuf = _NBUF
    nstep = num_rows // bkl
    main = nstep * bkl
    tail = num_rows - main
    ct = clip_embs.T
    tail_arr = ct[:, main:] if tail else jnp.zeros((d, 1), jnp.float32)
    tail_w = tail if tail else 1

    p_hi = prompt_embs.astype(jnp.bfloat16).astype(jnp.float32)
    p_lo = (prompt_embs - p_hi).astype(jnp.bfloat16).astype(jnp.float32)
    w1 = -2.0 * p_hi
    w2 = -2.0 * p_lo

    idx = pl.pallas_call(
        functools.partial(_nn_kernel, bkl=bkl, nbuf=nbuf, nstep=nstep,
                          num_rows=num_rows, tail=tail),
        grid=(nstep,),
        in_specs=[
            pl.BlockSpec((p, d), lambda i: (0, 0)),
            pl.BlockSpec((p, d), lambda i: (0, 0)),
            pl.BlockSpec((d, tail_w), lambda i: (0, 0)),
            pl.BlockSpec(memory_space=pl.ANY),
        ],
        out_specs=pl.BlockSpec((p, 1), lambda i: (0, 0)),
        out_shape=jax.ShapeDtypeStruct((p, 1), jnp.int32),
        scratch_shapes=[
            pltpu.VMEM((nbuf, d, bkl), jnp.float32),
            pltpu.SemaphoreType.DMA((nbuf,)),
            pltpu.VMEM((p, 1), jnp.float32),
            pltpu.VMEM((p, 1), jnp.int32),
        ],
    )(w1, w2, tail_arr, ct)

    ids = idx[:, 0]
    return (prompt_embs, prompt_embs, ids)


# speedup vs baseline: 11.2020x; 1.3127x over previous
import functools

import jax
import jax.numpy as jnp
from jax.experimental import pallas as pl
from jax.experimental.pallas import tpu as pltpu

_BKL = 31232
_HSUB = 4
_NBUF = 3


def _scores(w1, w2, ones, ct):
    ct_hi = ct.astype(jnp.bfloat16).astype(jnp.float32)
    ct_lo = ct - ct_hi
    sq = ct * ct
    sq_hi = sq.astype(jnp.bfloat16).astype(jnp.float32)
    sq_lo = sq - sq_hi

    def mm(a, b):
        return jax.lax.dot_general(
            a, b, (((1,), (0,)), ((), ())),
            preferred_element_type=jnp.float32)

    return (mm(w1, ct_hi) + mm(w1, ct_lo)) + (
        mm(w2, ct_hi) + (mm(ones, sq_hi) + mm(ones, sq_lo)))


def _argmin_lanes(scores, base, num_rows):
    local_min = jnp.min(scores, axis=1, keepdims=True)
    lane_ids = jax.lax.broadcasted_iota(jnp.int32, scores.shape, 1)
    masked = jnp.where(scores == local_min, lane_ids, num_rows)
    local_arg = jnp.min(masked, axis=1, keepdims=True)
    return local_min, base + local_arg


def _nn_kernel(w1_ref, w2_ref, tail_ref, ct_hbm, idx_ref, bufs, sems,
               val_s, idx_s, *, bkl, nbuf, nstep, num_rows, tail):
    i = pl.program_id(0)
    w1 = w1_ref[...]
    w2 = w2_ref[...]
    ones = jnp.ones_like(w1)

    @pl.when(i == 0)
    def _prologue():
        for b in range(min(nbuf, nstep)):
            pltpu.make_async_copy(
                ct_hbm.at[:, pl.ds(b * bkl, bkl)], bufs.at[b], sems.at[b]
            ).start()
        if tail:
            tv, ti = _argmin_lanes(
                _scores(w1, w2, ones, tail_ref[...]),
                nstep * bkl, num_rows)
            val_s[...] = tv
            idx_s[...] = ti
        else:
            val_s[...] = jnp.full_like(val_s, jnp.inf)
            idx_s[...] = jnp.zeros_like(idx_s)

    slot = jax.lax.rem(i, nbuf)
    pltpu.make_async_copy(
        ct_hbm.at[:, pl.ds(i * bkl, bkl)], bufs.at[slot], sems.at[slot]
    ).wait()

    hsub = _HSUB
    sub = bkl // hsub
    cbuf = bufs[slot]
    for h in range(hsub):
        local_min, local_idx = _argmin_lanes(
            _scores(w1, w2, ones, cbuf[:, h * sub:(h + 1) * sub]),
            i * bkl + h * sub, num_rows)

        prev_v = val_s[...]
        prev_i = idx_s[...]
        better = (local_min < prev_v) | (
            (local_min == prev_v) & (local_idx < prev_i))
        val_s[...] = jnp.where(better, local_min, prev_v)
        idx_s[...] = jnp.where(better, local_idx, prev_i)

    @pl.when(i + nbuf < nstep)
    def _refill():
        pltpu.make_async_copy(
            ct_hbm.at[:, pl.ds((i + nbuf) * bkl, bkl)], bufs.at[slot],
            sems.at[slot]
        ).start()

    @pl.when(i == nstep - 1)
    def _finish():
        idx_ref[...] = idx_s[...]


def kernel(prompt_embs, clip_embs):
    num_rows, d = clip_embs.shape
    p = prompt_embs.shape[0]
    bkl = _BKL
    nbuf = _NBUF
    nstep = num_rows // bkl
    main = nstep * bkl
    tail = num_rows - main
    ct = clip_embs.T
    tail_arr = ct[:, main:] if tail else jnp.zeros((d, 1), jnp.float32)
    tail_w = tail if tail else 1

    p_hi = prompt_embs.astype(jnp.bfloat16).astype(jnp.float32)
    p_lo = (prompt_embs - p_hi).astype(jnp.bfloat16).astype(jnp.float32)
    w1 = -2.0 * p_hi
    w2 = -2.0 * p_lo

    idx = pl.pallas_call(
        functools.partial(_nn_kernel, bkl=bkl, nbuf=nbuf, nstep=nstep,
                          num_rows=num_rows, tail=tail),
        grid=(nstep,),
        in_specs=[
            pl.BlockSpec((p, d), lambda i: (0, 0)),
            pl.BlockSpec((p, d), lambda i: (0, 0)),
            pl.BlockSpec((d, tail_w), lambda i: (0, 0)),
            pl.BlockSpec(memory_space=pl.ANY),
        ],
        out_specs=pl.BlockSpec((p, 1), lambda i: (0, 0)),
        out_shape=jax.ShapeDtypeStruct((p, 1), jnp.int32),
        scratch_shapes=[
            pltpu.VMEM((nbuf, d, bkl), jnp.float32),
            pltpu.SemaphoreType.DMA((nbuf,)),
            pltpu.VMEM((p, 1), jnp.float32),
            pltpu.VMEM((p, 1), jnp.int32),
        ],
    )(w1, w2, tail_arr, ct)

    ids = idx[:, 0]
    return (prompt_embs, prompt_embs, ids)
